# Initial kernel scaffold; baseline (speedup 1.0000x reference)
import jax
import jax.numpy as jnp
from jax.experimental import pallas as pl


def kernel(edge_index, edge_values, members, user_table, item_table, wc_w, wo_w):
    raise NotImplementedError("write your pallas kernel here")


# speedup vs baseline: 3.8328x; 3.8328x over previous
import functools

import jax
import jax.numpy as jnp
from jax import lax
from jax.experimental import pallas as pl
from jax.experimental.pallas import tpu as pltpu
from jax.experimental.pallas import tpu_sc as plsc

NUM_USERS = 5000
NUM_ITEMS = 5000
N = NUM_USERS + NUM_ITEMS
E = 320000
D = 128
N_LAYERS = 3
G = 1000
GS = 8

NC = 2
NS = 16
NW = NC * NS
L = 16

CH = 128
NCH = -(-E // (NW * CH))
EW = NCH * CH
NP = 10240
ROWS_PER_TILE = NP // NS

GP = NW * 32
GCH = 2

_mesh = plsc.VectorSubcoreMesh(
    core_axis_name="c", subcore_axis_name="s", num_cores=NC, num_subcores=NS
)


def _make_spmm():
  @functools.partial(
      pl.kernel,
      out_type=jax.ShapeDtypeStruct((NC, NP, D), jnp.float32),
      mesh=_mesh,
      scratch_types=[
          pltpu.VMEM((NCH, CH), jnp.int32),
          pltpu.VMEM((NCH, CH), jnp.int32),
          pltpu.VMEM((NCH, CH), jnp.float32),
          pltpu.VMEM((CH, D), jnp.float32),
          pltpu.VMEM_SHARED((NP, D), jnp.float32),
          pltpu.SemaphoreType.DMA,
      ],
  )
  def spmm(x_hbm, src_h, dst_h, val_h, out_h,
           sidx_v, didx_v, val_v, rows_v, acc_sh, gsem):
    ci = lax.axis_index("c")
    s = lax.axis_index("s")
    wid = ci * NS + s

    pltpu.sync_copy(src_h.at[wid], sidx_v)
    pltpu.sync_copy(dst_h.at[wid], didx_v)
    pltpu.sync_copy(val_h.at[wid], val_v)

    zv = jnp.zeros((L,), jnp.float32)

    @pl.loop(0, CH)
    def _(r):
      for j in range(D // L):
        rows_v[r, pl.ds(j * L, L)] = zv

    base = s * ROWS_PER_TILE
    nfull = ROWS_PER_TILE // CH
    for k in range(nfull):
      pltpu.sync_copy(rows_v, acc_sh.at[pl.ds(base + k * CH, CH)])
    plsc.subcore_barrier()

    @pl.loop(0, NCH)
    def _(c):
      pltpu.async_copy(x_hbm.at[sidx_v.at[c]], rows_v, gsem).wait()

      @pl.loop(0, CH, step=L)
      def _(e0):
        v16 = val_v[c, pl.ds(e0, L)]
        for i in range(L):
          bv = jnp.full((L,), v16[i], jnp.float32)
          for j in range(D // L):
            sl = pl.ds(j * L, L)
            rows_v[e0 + i, sl] = rows_v[e0 + i, sl] * bv

      pltpu.sync_copy(rows_v, acc_sh.at[didx_v.at[c]], add=True)

    plsc.subcore_barrier()

    for k in range(nfull):
      sl = pl.ds(base + k * CH, CH)
      pltpu.sync_copy(acc_sh.at[sl], out_h.at[ci, sl])

  return spmm


_spmm = _make_spmm()


def _make_grouppool():
  @functools.partial(
      pl.kernel,
      out_type=(
          jax.ShapeDtypeStruct((GP, D), jnp.float32),
          jax.ShapeDtypeStruct((GP, D), jnp.float32),
      ),
      mesh=_mesh,
      scratch_types=[
          pltpu.VMEM((GCH, CH), jnp.int32),
          pltpu.VMEM((CH, D), jnp.float32),
          pltpu.VMEM((CH // GS, D), jnp.float32),
          pltpu.VMEM((CH // GS, D), jnp.float32),
          pltpu.SemaphoreType.DMA,
      ],
  )
  def grouppool(emb_hbm, memb_h, mid_h, half_h,
                midx_v, rows_v, mid_v, half_v, gsem):
    ci = lax.axis_index("c")
    s = lax.axis_index("s")
    wid = ci * NS + s
    gpc = CH // GS

    pltpu.sync_copy(memb_h.at[wid], midx_v)
    for k in range(GCH):
      pltpu.async_copy(emb_hbm.at[midx_v.at[k]], rows_v, gsem).wait()

      @pl.loop(0, gpc)
      def _(g):
        r0 = g * GS
        for j in range(D // L):
          sl = pl.ds(j * L, L)
          mx = rows_v[r0, sl]
          mn = mx
          for m in range(1, GS):
            r = rows_v[r0 + m, sl]
            mx = jnp.maximum(mx, r)
            mn = jnp.minimum(mn, r)
          mid_v[g, sl] = (mx + mn) * 0.5
          half_v[g, sl] = (mx - mn) * 0.5

      obase = wid * (GCH * gpc) + k * gpc
      pltpu.sync_copy(mid_v, mid_h.at[pl.ds(obase, gpc)])
      pltpu.sync_copy(half_v, half_h.at[pl.ds(obase, gpc)])

  return grouppool


_grouppool = _make_grouppool()


def _combine(p01, accin):
  def body(p_ref, acc_ref, x_ref, accout_ref):
    xv = p_ref[0] + p_ref[1]
    x_ref[...] = xv
    accout_ref[...] = acc_ref[...] + xv

  nb = 10
  rb = N // nb
  return pl.pallas_call(
      body,
      grid=(nb,),
      in_specs=[
          pl.BlockSpec((2, rb, D), lambda i: (0, i, 0)),
          pl.BlockSpec((rb, D), lambda i: (i, 0)),
      ],
      out_specs=[
          pl.BlockSpec((rb, D), lambda i: (i, 0)),
          pl.BlockSpec((rb, D), lambda i: (i, 0)),
      ],
      out_shape=(
          jax.ShapeDtypeStruct((N, D), jnp.float32),
          jax.ShapeDtypeStruct((N, D), jnp.float32),
      ),
  )(p01, accin)


def _finalize(p01, accin):
  def body(p_ref, acc_ref, emb_ref):
    emb_ref[...] = (acc_ref[...] + p_ref[0] + p_ref[1]) * 0.25

  nb = 10
  rb = N // nb
  return pl.pallas_call(
      body,
      grid=(nb,),
      in_specs=[
          pl.BlockSpec((2, rb, D), lambda i: (0, i, 0)),
          pl.BlockSpec((rb, D), lambda i: (i, 0)),
      ],
      out_specs=pl.BlockSpec((rb, D), lambda i: (i, 0)),
      out_shape=jax.ShapeDtypeStruct((N, D), jnp.float32),
  )(p01, accin)


def _groupmm(mid, half, wc_w, wo_w):
  def body(m_ref, h_ref, wc_ref, wo_ref, c_ref, o_ref):
    dn = (((1,), (1,)), ((), ()))
    c_ref[...] = lax.dot_general(
        m_ref[...], wc_ref[...], dn,
        precision=lax.Precision.HIGHEST,
        preferred_element_type=jnp.float32)
    o_ref[...] = lax.dot_general(
        h_ref[...], wo_ref[...], dn,
        precision=lax.Precision.HIGHEST,
        preferred_element_type=jnp.float32)

  return pl.pallas_call(
      body,
      out_shape=(
          jax.ShapeDtypeStruct((GP, D), jnp.float32),
          jax.ShapeDtypeStruct((GP, D), jnp.float32),
      ),
  )(mid, half, wc_w, wo_w)


@jax.jit
def kernel(edge_index, edge_values, members, user_table, item_table, wc_w, wo_w):
  x0 = jnp.concatenate([user_table, item_table], axis=0)

  dst = edge_index[0]
  src = edge_index[1]
  epad = NW * EW - E
  srcp = jnp.concatenate([src, jnp.zeros((epad,), jnp.int32)]).reshape(NW, NCH, CH)
  dstp = jnp.concatenate([dst, jnp.zeros((epad,), jnp.int32)]).reshape(NW, NCH, CH)
  valp = jnp.concatenate(
      [edge_values, jnp.zeros((epad,), jnp.float32)]).reshape(NW, NCH, CH)

  x = x0
  acc = x0
  for layer in range(N_LAYERS):
    p01 = _spmm(x, srcp, dstp, valp)
    if layer < N_LAYERS - 1:
      x, acc = _combine(p01, acc)
    else:
      emb = _finalize(p01, acc)

  mpad = GP * GS - G * GS
  membp = jnp.concatenate(
      [members.reshape(-1), jnp.zeros((mpad,), jnp.int32)]).reshape(NW, GCH, CH)
  mid, half = _grouppool(emb, membp)
  centers, offsets = _groupmm(mid, half, wc_w, wo_w)

  return (emb[:NUM_USERS], emb[NUM_USERS:], centers[:G], offsets[:G])
